# baseline (speedup 1.0000x reference)
import functools

import jax
import jax.numpy as jnp
from jax import lax
from jax.experimental import pallas as pl
from jax.experimental.pallas import tpu as pltpu
from jax.experimental.pallas import tpu_sc as plsc

_NC = 2
_NS = 16
_NW = _NC * _NS
_L = 16


def _gather_fields(idx_v, fidx_bufs, g_bufs, tbl_refs, sem, fields,
                   num_fields, nchunks):
  iota = lax.iota(jnp.int32, _L)
  iota_f = iota * num_fields

  for i, f in enumerate(fields):
    def j_body(j, _, f=f, i=i):
      p = iota_f + (j * _L * num_fields + f)
      fidx_bufs[i][pl.ds(j * _L, _L)] = plsc.load_gather(
          idx_v, [p >> 7, p & 127])
      return 0

    lax.fori_loop(0, nchunks, j_body, 0)
    pltpu.async_copy(tbl_refs[i].at[fidx_bufs[i]], g_bufs[i], sem)

  for i in range(len(fields)):
    pltpu.make_async_copy(
        tbl_refs[i].at[fidx_bufs[i]], g_bufs[i], sem).wait()


def _lr_body_a(*refs, bpw, num_fields, fields):
  nf = len(fields)
  idx_hbm = refs[0]
  tbl_refs = refs[1:1 + nf]
  out_hbm = refs[1 + nf]
  rest = refs[2 + nf:]
  idx_v, out_v = rest[:2]
  fidx_bufs = rest[2:2 + nf]
  g_bufs = rest[2 + nf:2 + 2 * nf]
  sem = rest[2 + 2 * nf]

  wid = lax.axis_index("s") * _NC + lax.axis_index("c")
  nrows = (bpw * num_fields) // 128
  pltpu.sync_copy(idx_hbm.at[pl.ds(wid * nrows, nrows), :], idx_v)
  nchunks = bpw // _L
  _gather_fields(idx_v, fidx_bufs, g_bufs, tbl_refs, sem, fields,
                 num_fields, nchunks)

  def chunk_body(j, _):
    sl = pl.ds(j * _L, _L)
    acc = g_bufs[0][sl]
    for i in range(1, nf):
      acc = acc + g_bufs[i][sl]
    out_v[sl] = acc
    return 0

  lax.fori_loop(0, nchunks, chunk_body, 0)
  pltpu.sync_copy(out_v, out_hbm.at[pl.ds(wid * bpw, bpw)])


def _lr_body_b(*refs, bpw, num_fields, fields):
  nf = len(fields)
  idx_hbm, part_hbm, num_hbm = refs[:3]
  tbl_refs = refs[3:3 + nf]
  out_hbm = refs[3 + nf]
  rest = refs[4 + nf:]
  idx_v, part_v, num_v, out_v = rest[:4]
  fidx_bufs = rest[4:4 + nf]
  g_bufs = rest[4 + nf:4 + 2 * nf]
  sem = rest[4 + 2 * nf]

  wid = lax.axis_index("s") * _NC + lax.axis_index("c")
  nrows = (bpw * num_fields) // 128
  pltpu.sync_copy(idx_hbm.at[pl.ds(wid * nrows, nrows), :], idx_v)
  pltpu.sync_copy(part_hbm.at[pl.ds(wid * bpw, bpw)], part_v)
  pltpu.sync_copy(num_hbm.at[0, pl.ds(wid * bpw, bpw)], num_v)
  nchunks = bpw // _L
  _gather_fields(idx_v, fidx_bufs, g_bufs, tbl_refs, sem, fields,
                 num_fields, nchunks)

  def chunk_body(j, _):
    sl = pl.ds(j * _L, _L)
    acc = part_v[sl] + num_v[sl]
    for i in range(nf):
      acc = acc + g_bufs[i][sl]
    out_v[sl] = acc
    return 0

  lax.fori_loop(0, nchunks, chunk_body, 0)
  pltpu.sync_copy(out_v, out_hbm.at[pl.ds(wid * bpw, bpw)])


def _lin_body(xn_ref, w_ref, b_ref, o_ref):
  o_ref[...] = jax.lax.dot_general(
      w_ref[...], xn_ref[...], (((1,), (1,)), ((), ())),
      preferred_element_type=jnp.float32) + b_ref[0]


@functools.partial(jax.jit, static_argnames=())
def kernel(x_cat, x_num, tables, W, bias):
  B, F = x_cat.shape
  _, D_NUM = x_num.shape
  bpw = B // _NW
  fa = tuple(range(16))
  fb = tuple(range(16, F))

  idx = x_cat.reshape(B * F // 128, 128)
  idx, tables_a = lax.optimization_barrier((idx, tables))
  sl_a = [tables_a[f, :, 0] for f in fa]
  tables_b, sl_a0, xn_b = lax.optimization_barrier((tables, sl_a[0], x_num))
  sl_a = [sl_a0] + sl_a[1:]
  sl_b = [tables_b[f, :, 0] for f in fb]

  num = pl.pallas_call(
      _lin_body,
      out_shape=jax.ShapeDtypeStruct((1, B), jnp.float32),
  )(xn_b, W, bias)

  mesh = plsc.VectorSubcoreMesh(core_axis_name="c", subcore_axis_name="s",
                                num_cores=_NC, num_subcores=_NS)
  cp = pltpu.CompilerParams(needs_layout_passes=False)

  body_a = functools.partial(_lr_body_a, bpw=bpw, num_fields=F, fields=fa)
  part = pl.kernel(
      body_a,
      out_type=jax.ShapeDtypeStruct((B,), jnp.float32),
      mesh=mesh,
      compiler_params=cp,
      scratch_types=(
          [pltpu.VMEM((bpw * F // 128, 128), jnp.int32),
           pltpu.VMEM((bpw,), jnp.float32)]
          + [pltpu.VMEM((bpw,), jnp.int32) for _ in fa]
          + [pltpu.VMEM((bpw,), jnp.float32) for _ in fa]
          + [pltpu.SemaphoreType.DMA]
      ),
  )(idx, *sl_a)

  body_b = functools.partial(_lr_body_b, bpw=bpw, num_fields=F, fields=fb)
  out = pl.kernel(
      body_b,
      out_type=jax.ShapeDtypeStruct((B,), jnp.float32),
      mesh=mesh,
      compiler_params=cp,
      scratch_types=(
          [pltpu.VMEM((bpw * F // 128, 128), jnp.int32),
           pltpu.VMEM((bpw,), jnp.float32),
           pltpu.VMEM((bpw,), jnp.float32),
           pltpu.VMEM((bpw,), jnp.float32)]
          + [pltpu.VMEM((bpw,), jnp.int32) for _ in fb]
          + [pltpu.VMEM((bpw,), jnp.float32) for _ in fb]
          + [pltpu.SemaphoreType.DMA]
      ),
  )(idx, part, num, *sl_b)
  return out.reshape(B, 1)


# speedup vs baseline: 1.0165x; 1.0165x over previous
import functools

import jax
import jax.numpy as jnp
from jax import lax
from jax.experimental import pallas as pl
from jax.experimental.pallas import tpu as pltpu
from jax.experimental.pallas import tpu_sc as plsc

_NC = 2
_NS = 16
_NW = _NC * _NS
_L = 16


def _gather_fields(idx_v, fidx_bufs, g_bufs, tbl_refs, sem, fields,
                   num_fields, nchunks):
  iota = lax.iota(jnp.int32, _L)
  iota_f = iota * num_fields

  for i, f in enumerate(fields):
    def j_body(j, _, f=f, i=i):
      fidx_bufs[i][pl.ds(j * _L, _L)] = plsc.load_gather(
          idx_v, [iota_f + (j * _L * num_fields + f)])
      return 0

    lax.fori_loop(0, nchunks, j_body, 0)
    pltpu.async_copy(tbl_refs[i].at[fidx_bufs[i]], g_bufs[i], sem)

  for i in range(len(fields)):
    pltpu.make_async_copy(
        tbl_refs[i].at[fidx_bufs[i]], g_bufs[i], sem).wait()


def _lr_body_a(*refs, bpw, num_fields, fields):
  nf = len(fields)
  idx_hbm = refs[0]
  tbl_refs = refs[1:1 + nf]
  out_hbm = refs[1 + nf]
  rest = refs[2 + nf:]
  idx_v, out_v = rest[:2]
  fidx_bufs = rest[2:2 + nf]
  g_bufs = rest[2 + nf:2 + 2 * nf]
  sem = rest[2 + 2 * nf]

  wid = lax.axis_index("s") * _NC + lax.axis_index("c")
  pltpu.sync_copy(idx_hbm.at[wid], idx_v)
  nchunks = bpw // _L
  _gather_fields(idx_v, fidx_bufs, g_bufs, tbl_refs, sem, fields,
                 num_fields, nchunks)

  def chunk_body(j, _):
    sl = pl.ds(j * _L, _L)
    acc = g_bufs[0][sl]
    for i in range(1, nf):
      acc = acc + g_bufs[i][sl]
    out_v[sl] = acc
    return 0

  lax.fori_loop(0, nchunks, chunk_body, 0)
  pltpu.sync_copy(out_v, out_hbm.at[pl.ds(wid * bpw, bpw)])


def _lr_body_b(*refs, bpw, num_fields, fields):
  nf = len(fields)
  idx_hbm, part_hbm, num_hbm = refs[:3]
  tbl_refs = refs[3:3 + nf]
  out_hbm = refs[3 + nf]
  rest = refs[4 + nf:]
  idx_v, part_v, num_v, out_v = rest[:4]
  fidx_bufs = rest[4:4 + nf]
  g_bufs = rest[4 + nf:4 + 2 * nf]
  sem = rest[4 + 2 * nf]

  wid = lax.axis_index("s") * _NC + lax.axis_index("c")
  pltpu.sync_copy(idx_hbm.at[wid], idx_v)
  pltpu.sync_copy(part_hbm.at[pl.ds(wid * bpw, bpw)], part_v)
  pltpu.sync_copy(num_hbm.at[0, pl.ds(wid * bpw, bpw)], num_v)
  nchunks = bpw // _L
  _gather_fields(idx_v, fidx_bufs, g_bufs, tbl_refs, sem, fields,
                 num_fields, nchunks)

  def chunk_body(j, _):
    sl = pl.ds(j * _L, _L)
    acc = part_v[sl] + num_v[sl]
    for i in range(nf):
      acc = acc + g_bufs[i][sl]
    out_v[sl] = acc
    return 0

  lax.fori_loop(0, nchunks, chunk_body, 0)
  pltpu.sync_copy(out_v, out_hbm.at[pl.ds(wid * bpw, bpw)])


def _lin_body(xn_ref, w_ref, b_ref, o_ref):
  o_ref[...] = jax.lax.dot_general(
      w_ref[...], xn_ref[...], (((1,), (1,)), ((), ())),
      preferred_element_type=jnp.float32) + b_ref[0]


@functools.partial(jax.jit, static_argnames=())
def kernel(x_cat, x_num, tables, W, bias):
  B, F = x_cat.shape
  _, D_NUM = x_num.shape
  bpw = B // _NW
  fa = tuple(range(16))
  fb = tuple(range(16, F))

  idx = x_cat.reshape(_NW, bpw * F)
  idx, tables_a = lax.optimization_barrier((idx, tables))
  sl_a = [tables_a[f, :, 0] for f in fa]
  tables_b, sl_a0, xn_b = lax.optimization_barrier((tables, sl_a[0], x_num))
  sl_a = [sl_a0] + sl_a[1:]
  sl_b = [tables_b[f, :, 0] for f in fb]

  num = pl.pallas_call(
      _lin_body,
      out_shape=jax.ShapeDtypeStruct((1, B), jnp.float32),
  )(xn_b, W, bias)

  mesh = plsc.VectorSubcoreMesh(core_axis_name="c", subcore_axis_name="s",
                                num_cores=_NC, num_subcores=_NS)
  cp = pltpu.CompilerParams(needs_layout_passes=False)

  body_a = functools.partial(_lr_body_a, bpw=bpw, num_fields=F, fields=fa)
  part = pl.kernel(
      body_a,
      out_type=jax.ShapeDtypeStruct((B,), jnp.float32),
      mesh=mesh,
      compiler_params=cp,
      scratch_types=(
          [pltpu.VMEM((bpw * F,), jnp.int32),
           pltpu.VMEM((bpw,), jnp.float32)]
          + [pltpu.VMEM((bpw,), jnp.int32) for _ in fa]
          + [pltpu.VMEM((bpw,), jnp.float32) for _ in fa]
          + [pltpu.SemaphoreType.DMA]
      ),
  )(idx, *sl_a)

  body_b = functools.partial(_lr_body_b, bpw=bpw, num_fields=F, fields=fb)
  out = pl.kernel(
      body_b,
      out_type=jax.ShapeDtypeStruct((B,), jnp.float32),
      mesh=mesh,
      compiler_params=cp,
      scratch_types=(
          [pltpu.VMEM((bpw * F,), jnp.int32),
           pltpu.VMEM((bpw,), jnp.float32),
           pltpu.VMEM((bpw,), jnp.float32),
           pltpu.VMEM((bpw,), jnp.float32)]
          + [pltpu.VMEM((bpw,), jnp.int32) for _ in fb]
          + [pltpu.VMEM((bpw,), jnp.float32) for _ in fb]
          + [pltpu.SemaphoreType.DMA]
      ),
  )(idx, part, num, *sl_b)
  return out.reshape(B, 1)
